# Initial kernel scaffold; baseline (speedup 1.0000x reference)
import jax
import jax.numpy as jnp
from jax.experimental import pallas as pl


def kernel(text_repr, label_repr, image_repr, W0, as0, ad0, b0, g0, be0, W1, as1, ad1, b1, g1, be1, W2, as2, ad2, b2, g2, be2):
    raise NotImplementedError("write your pallas kernel here")


# speedup vs baseline: 113.6818x; 113.6818x over previous
import functools

import jax
import jax.numpy as jnp
from jax.experimental import pallas as pl

HEADS = 4
TOPK = 3
NEG_SLOPE = 0.2


def _rownorm(v):
    n = jnp.sqrt(jnp.sum(v * v, axis=-1, keepdims=True))
    return v / jnp.maximum(n, 1e-8)


def _topk_mask(sim, L):
    cols = [sim[:, j:j + 1] for j in range(L)]
    outs = []
    for j in range(L):
        r = jnp.zeros_like(cols[0])
        for k in range(L):
            if k == j:
                continue
            if k < j:
                beat = cols[k] >= cols[j]
            else:
                beat = cols[k] > cols[j]
            r = r + beat.astype(jnp.float32)
        outs.append((r < (TOPK - 0.5)).astype(jnp.float32))
    return jnp.concatenate(outs, axis=1)


def _edges_kernel(t_ref, l_ref, i_ref, a_ref, *, T, L, I):
    t = t_ref[0]
    lab = l_ref[0]
    img = i_ref[0]
    tn = _rownorm(t)
    labn = _rownorm(lab)
    imgn = _rownorm(img)
    dn = (((1,), (1,)), ((), ()))
    sim_l = jax.lax.dot_general(tn, labn, dn, preferred_element_type=jnp.float32)
    sim_i = jax.lax.dot_general(tn, imgn, dn, preferred_element_type=jnp.float32)
    mask_l = _topk_mask(sim_l, L)
    mask_i = _topk_mask(sim_i, I)
    eye_l = (jax.lax.broadcasted_iota(jnp.int32, (L, L), 0)
             == jax.lax.broadcasted_iota(jnp.int32, (L, L), 1)).astype(jnp.float32)
    mask_lT = jax.lax.dot_general(eye_l, mask_l, dn,
                                  preferred_element_type=jnp.float32)
    eye_i = (jax.lax.broadcasted_iota(jnp.int32, (I, I), 0)
             == jax.lax.broadcasted_iota(jnp.int32, (I, I), 1)).astype(jnp.float32)
    mask_iT = jax.lax.dot_general(eye_i, mask_i, dn,
                                  preferred_element_type=jnp.float32)

    r = jax.lax.broadcasted_iota(jnp.int32, (T, T), 0)
    c = jax.lax.broadcasted_iota(jnp.int32, (T, T), 1)
    chain = ((r - c == 1) | (c - r == 1) | (r == c)).astype(jnp.float32)

    ones_ll = jnp.ones((L, L), jnp.float32)
    ones_ii = jnp.ones((I, I), jnp.float32)
    ones_il = jnp.ones((I, L), jnp.float32)
    ones_li = jnp.ones((L, I), jnp.float32)

    a_ref[0, 0:T, 0:T] = chain
    a_ref[0, 0:T, T:T + L] = mask_l
    a_ref[0, 0:T, T + L:T + L + I] = mask_i
    a_ref[0, T:T + L, 0:T] = mask_lT
    a_ref[0, T:T + L, T:T + L] = ones_ll + eye_l
    a_ref[0, T:T + L, T + L:T + L + I] = ones_li
    a_ref[0, T + L:T + L + I, 0:T] = mask_iT
    a_ref[0, T + L:T + L + I, T:T + L] = ones_il
    a_ref[0, T + L:T + L + I, T + L:T + L + I] = ones_ii + eye_i


def _layer_kernel(x_ref, a_ref, w_ref, as_ref, ad_ref, b_ref, g_ref, be_ref,
                  o_ref, *, N, H):
    out_ch = H // HEADS
    x = x_ref[0]
    A = a_ref[0]
    h = jnp.dot(x, w_ref[...], preferred_element_type=jnp.float32)
    dn = (((1,), (1,)), ((), ()))
    aggs = []
    for hd in range(HEADS):
        hh = h[:, hd * out_ch:(hd + 1) * out_ch]
        asr = as_ref[hd:hd + 1, :]
        adr = ad_ref[hd:hd + 1, :]
        a_src = jax.lax.dot_general(asr, hh, dn,
                                    preferred_element_type=jnp.float32)
        a_dst = jax.lax.dot_general(hh, adr, dn,
                                    preferred_element_type=jnp.float32)
        alpha = a_dst + a_src
        alpha = jnp.where(alpha >= 0, alpha, NEG_SLOPE * alpha)
        malpha = jnp.where(A > 0, alpha, -1e30)
        amax = jnp.max(malpha, axis=1, keepdims=True)
        ex = A * jnp.exp(jnp.minimum(alpha - amax, 0.0))
        den = jnp.sum(ex, axis=1, keepdims=True)
        P = ex / (den + 1e-16)
        aggs.append(jnp.dot(P, hh, preferred_element_type=jnp.float32))
    agg = jnp.concatenate(aggs, axis=1)
    out = jnp.maximum(agg + b_ref[...], 0.0)
    y = out + x
    mu = jnp.mean(y, axis=1, keepdims=True)
    yc = y - mu
    var = jnp.mean(yc * yc, axis=1, keepdims=True)
    o_ref[0] = yc / jnp.sqrt(var + 1e-5) * g_ref[...] + be_ref[...]


def _build_A(text, label, image, interpret=False):
    B, T, H = text.shape
    L = label.shape[1]
    I = image.shape[1]
    N = T + L + I
    return pl.pallas_call(
        functools.partial(_edges_kernel, T=T, L=L, I=I),
        grid=(B,),
        in_specs=[
            pl.BlockSpec((1, T, H), lambda b: (b, 0, 0)),
            pl.BlockSpec((1, L, H), lambda b: (b, 0, 0)),
            pl.BlockSpec((1, I, H), lambda b: (b, 0, 0)),
        ],
        out_specs=pl.BlockSpec((1, N, N), lambda b: (b, 0, 0)),
        out_shape=jax.ShapeDtypeStruct((B, N, N), jnp.float32),
        interpret=interpret,
    )(text, label, image)


def _layer(x, A, W, a_s, a_d, b, g, be, interpret=False):
    B, N, H = x.shape
    out_ch = H // HEADS
    return pl.pallas_call(
        functools.partial(_layer_kernel, N=N, H=H),
        grid=(B,),
        in_specs=[
            pl.BlockSpec((1, N, H), lambda b: (b, 0, 0)),
            pl.BlockSpec((1, N, N), lambda b: (b, 0, 0)),
            pl.BlockSpec((H, H), lambda b: (0, 0)),
            pl.BlockSpec((HEADS, out_ch), lambda b: (0, 0)),
            pl.BlockSpec((HEADS, out_ch), lambda b: (0, 0)),
            pl.BlockSpec((1, H), lambda b: (0, 0)),
            pl.BlockSpec((1, H), lambda b: (0, 0)),
            pl.BlockSpec((1, H), lambda b: (0, 0)),
        ],
        out_specs=pl.BlockSpec((1, N, H), lambda b: (b, 0, 0)),
        out_shape=jax.ShapeDtypeStruct((B, N, H), jnp.float32),
        interpret=interpret,
    )(x, A, W, a_s, a_d, b, g, be)


def _run(text_repr, label_repr, image_repr, params, interpret=False):
    B, T, H = text_repr.shape
    x = jnp.concatenate([text_repr, label_repr, image_repr], axis=1)
    A = _build_A(text_repr, label_repr, image_repr, interpret=interpret)
    for (W, a_s, a_d, b, g, be) in params:
        x = _layer(x, A, W, a_s, a_d, b.reshape(1, -1), g.reshape(1, -1),
                   be.reshape(1, -1), interpret=interpret)
    return x[:, :T, :]


def kernel(text_repr, label_repr, image_repr,
           W0, as0, ad0, b0, g0, be0,
           W1, as1, ad1, b1, g1, be1,
           W2, as2, ad2, b2, g2, be2):
    params = [
        (W0, as0, ad0, b0, g0, be0),
        (W1, as1, ad1, b1, g1, be1),
        (W2, as2, ad2, b2, g2, be2),
    ]
    return _run(text_repr, label_repr, image_repr, params)


# speedup vs baseline: 162.3256x; 1.4279x over previous
import functools

import jax
import jax.numpy as jnp
from jax.experimental import pallas as pl
from jax.experimental.pallas import tpu as pltpu

HEADS = 4
TOPK = 3
NEG_SLOPE = 0.2


def _rownorm(v):
    n = jnp.sqrt(jnp.sum(v * v, axis=-1, keepdims=True))
    return v / jnp.maximum(n, 1e-8)


def _topk_mask(sim, L):
    cols = [sim[:, j:j + 1] for j in range(L)]
    outs = []
    for j in range(L):
        r = jnp.zeros_like(cols[0])
        for k in range(L):
            if k == j:
                continue
            if k < j:
                beat = cols[k] >= cols[j]
            else:
                beat = cols[k] > cols[j]
            r = r + beat.astype(jnp.float32)
        outs.append((r < (TOPK - 0.5)).astype(jnp.float32))
    return jnp.concatenate(outs, axis=1)


def _edges_kernel(t_ref, l_ref, i_ref, a_ref, *, T, L, I):
    t = t_ref[0]
    lab = l_ref[0]
    img = i_ref[0]
    tn = _rownorm(t)
    labn = _rownorm(lab)
    imgn = _rownorm(img)
    dn = (((1,), (1,)), ((), ()))
    sim_l = jax.lax.dot_general(tn, labn, dn, preferred_element_type=jnp.float32)
    sim_i = jax.lax.dot_general(tn, imgn, dn, preferred_element_type=jnp.float32)
    mask_l = _topk_mask(sim_l, L)
    mask_i = _topk_mask(sim_i, I)
    eye_l = (jax.lax.broadcasted_iota(jnp.int32, (L, L), 0)
             == jax.lax.broadcasted_iota(jnp.int32, (L, L), 1)).astype(jnp.float32)
    mask_lT = jax.lax.dot_general(eye_l, mask_l, dn,
                                  preferred_element_type=jnp.float32)
    eye_i = (jax.lax.broadcasted_iota(jnp.int32, (I, I), 0)
             == jax.lax.broadcasted_iota(jnp.int32, (I, I), 1)).astype(jnp.float32)
    mask_iT = jax.lax.dot_general(eye_i, mask_i, dn,
                                  preferred_element_type=jnp.float32)

    r = jax.lax.broadcasted_iota(jnp.int32, (T, T), 0)
    c = jax.lax.broadcasted_iota(jnp.int32, (T, T), 1)
    chain = ((r - c == 1) | (c - r == 1) | (r == c)).astype(jnp.float32)

    ones_ll = jnp.ones((L, L), jnp.float32)
    ones_ii = jnp.ones((I, I), jnp.float32)
    ones_il = jnp.ones((I, L), jnp.float32)
    ones_li = jnp.ones((L, I), jnp.float32)

    a_ref[0, 0:T, 0:T] = chain
    a_ref[0, 0:T, T:T + L] = mask_l
    a_ref[0, 0:T, T + L:T + L + I] = mask_i
    a_ref[0, T:T + L, 0:T] = mask_lT
    a_ref[0, T:T + L, T:T + L] = ones_ll + eye_l
    a_ref[0, T:T + L, T + L:T + L + I] = ones_li
    a_ref[0, T + L:T + L + I, 0:T] = mask_iT
    a_ref[0, T + L:T + L + I, T:T + L] = ones_il
    a_ref[0, T + L:T + L + I, T + L:T + L + I] = ones_ii + eye_i


def _layer_kernel(x_ref, a_ref, w_ref, as_ref, ad_ref, b_ref, g_ref, be_ref,
                  o_ref, *, N, H):
    out_ch = H // HEADS
    x = x_ref[0]
    A = a_ref[0]
    h = jnp.dot(x, w_ref[...], preferred_element_type=jnp.float32)
    dn = (((1,), (1,)), ((), ()))
    aggs = []
    for hd in range(HEADS):
        hh = h[:, hd * out_ch:(hd + 1) * out_ch]
        asr = as_ref[hd:hd + 1, :]
        adr = ad_ref[hd:hd + 1, :]
        a_src = jax.lax.dot_general(asr, hh, dn,
                                    preferred_element_type=jnp.float32)
        a_dst = jax.lax.dot_general(hh, adr, dn,
                                    preferred_element_type=jnp.float32)
        alpha = a_dst + a_src
        alpha = jnp.where(alpha >= 0, alpha, NEG_SLOPE * alpha)
        malpha = jnp.where(A > 0, alpha, -1e30)
        amax = jnp.max(malpha, axis=1, keepdims=True)
        ex = A * jnp.exp(jnp.minimum(alpha - amax, 0.0))
        den = jnp.sum(ex, axis=1, keepdims=True)
        P = ex / (den + 1e-16)
        aggs.append(jnp.dot(P, hh, preferred_element_type=jnp.float32))
    agg = jnp.concatenate(aggs, axis=1)
    out = jnp.maximum(agg + b_ref[...], 0.0)
    y = out + x
    mu = jnp.mean(y, axis=1, keepdims=True)
    yc = y - mu
    var = jnp.mean(yc * yc, axis=1, keepdims=True)
    o_ref[0] = yc / jnp.sqrt(var + 1e-5) * g_ref[...] + be_ref[...]


def _build_A(text, label, image, interpret=False):
    B, T, H = text.shape
    L = label.shape[1]
    I = image.shape[1]
    N = T + L + I
    return pl.pallas_call(
        functools.partial(_edges_kernel, T=T, L=L, I=I),
        grid=(B,),
        in_specs=[
            pl.BlockSpec((1, T, H), lambda b: (b, 0, 0)),
            pl.BlockSpec((1, L, H), lambda b: (b, 0, 0)),
            pl.BlockSpec((1, I, H), lambda b: (b, 0, 0)),
        ],
        out_specs=pl.BlockSpec((1, N, N), lambda b: (b, 0, 0)),
        out_shape=jax.ShapeDtypeStruct((B, N, N), jnp.float32),
        interpret=interpret,
    )(text, label, image)


def _layer(x, A, W, a_s, a_d, b, g, be, interpret=False):
    B, N, H = x.shape
    out_ch = H // HEADS
    return pl.pallas_call(
        functools.partial(_layer_kernel, N=N, H=H),
        grid=(B,),
        in_specs=[
            pl.BlockSpec((1, N, H), lambda b: (b, 0, 0)),
            pl.BlockSpec((1, N, N), lambda b: (b, 0, 0)),
            pl.BlockSpec((H, H), lambda b: (0, 0)),
            pl.BlockSpec((HEADS, out_ch), lambda b: (0, 0)),
            pl.BlockSpec((HEADS, out_ch), lambda b: (0, 0)),
            pl.BlockSpec((1, H), lambda b: (0, 0)),
            pl.BlockSpec((1, H), lambda b: (0, 0)),
            pl.BlockSpec((1, H), lambda b: (0, 0)),
        ],
        out_specs=pl.BlockSpec((1, N, H), lambda b: (b, 0, 0)),
        out_shape=jax.ShapeDtypeStruct((B, N, H), jnp.float32),
        interpret=interpret,
    )(x, A, W, a_s, a_d, b, g, be)


def _edge_mask(t, lab, img, T, L, I):
    tn = _rownorm(t)
    labn = _rownorm(lab)
    imgn = _rownorm(img)
    dn = (((1,), (1,)), ((), ()))
    sim_l = jax.lax.dot_general(tn, labn, dn, preferred_element_type=jnp.float32)
    sim_i = jax.lax.dot_general(tn, imgn, dn, preferred_element_type=jnp.float32)
    mask_l = _topk_mask(sim_l, L)
    mask_i = _topk_mask(sim_i, I)
    eye_l = (jax.lax.broadcasted_iota(jnp.int32, (L, L), 0)
             == jax.lax.broadcasted_iota(jnp.int32, (L, L), 1)).astype(jnp.float32)
    mask_lT = jax.lax.dot_general(eye_l, mask_l, dn,
                                  preferred_element_type=jnp.float32)
    eye_i = (jax.lax.broadcasted_iota(jnp.int32, (I, I), 0)
             == jax.lax.broadcasted_iota(jnp.int32, (I, I), 1)).astype(jnp.float32)
    mask_iT = jax.lax.dot_general(eye_i, mask_i, dn,
                                  preferred_element_type=jnp.float32)
    r = jax.lax.broadcasted_iota(jnp.int32, (T, T), 0)
    c = jax.lax.broadcasted_iota(jnp.int32, (T, T), 1)
    chain = ((r - c == 1) | (c - r == 1) | (r == c)).astype(jnp.float32)
    return chain, mask_l, mask_i, mask_lT, mask_iT, eye_l, eye_i


def _gat_layer(x, A, W, a_src_w, a_dst_w, b, g, be, N, H):
    out_ch = H // HEADS
    h = jnp.dot(x, W, preferred_element_type=jnp.float32)
    dn = (((1,), (1,)), ((), ()))
    aggs = []
    for hd in range(HEADS):
        hh = h[:, hd * out_ch:(hd + 1) * out_ch]
        asr = a_src_w[hd:hd + 1, :]
        adr = a_dst_w[hd:hd + 1, :]
        a_s = jax.lax.dot_general(asr, hh, dn,
                                  preferred_element_type=jnp.float32)
        a_d = jax.lax.dot_general(hh, adr, dn,
                                  preferred_element_type=jnp.float32)
        alpha = a_d + a_s
        alpha = jnp.where(alpha >= 0, alpha, NEG_SLOPE * alpha)
        malpha = jnp.where(A > 0, alpha, -1e30)
        amax = jnp.max(malpha, axis=1, keepdims=True)
        ex = A * jnp.exp(jnp.minimum(alpha - amax, 0.0))
        den = jnp.sum(ex, axis=1, keepdims=True)
        P = ex / (den + 1e-16)
        aggs.append(jnp.dot(P, hh, preferred_element_type=jnp.float32))
    agg = jnp.concatenate(aggs, axis=1)
    out = jnp.maximum(agg + b, 0.0)
    y = out + x
    mu = jnp.mean(y, axis=1, keepdims=True)
    yc = y - mu
    var = jnp.mean(yc * yc, axis=1, keepdims=True)
    return yc / jnp.sqrt(var + 1e-5) * g + be


def _fused_kernel(t_ref, l_ref, i_ref,
                  w0_ref, as0_ref, ad0_ref, b0_ref, g0_ref, be0_ref,
                  w1_ref, as1_ref, ad1_ref, b1_ref, g1_ref, be1_ref,
                  w2_ref, as2_ref, ad2_ref, b2_ref, g2_ref, be2_ref,
                  o_ref, a_scr, *, T, L, I, H):
    N = T + L + I
    t = t_ref[0]
    lab = l_ref[0]
    img = i_ref[0]
    chain, mask_l, mask_i, mask_lT, mask_iT, eye_l, eye_i = _edge_mask(
        t, lab, img, T, L, I)
    ones_ll = jnp.ones((L, L), jnp.float32)
    ones_ii = jnp.ones((I, I), jnp.float32)
    a_scr[0:T, 0:T] = chain
    a_scr[0:T, T:T + L] = mask_l
    a_scr[0:T, T + L:N] = mask_i
    a_scr[T:T + L, 0:T] = mask_lT
    a_scr[T:T + L, T:T + L] = ones_ll + eye_l
    a_scr[T:T + L, T + L:N] = jnp.ones((L, I), jnp.float32)
    a_scr[T + L:N, 0:T] = mask_iT
    a_scr[T + L:N, T:T + L] = jnp.ones((I, L), jnp.float32)
    a_scr[T + L:N, T + L:N] = ones_ii + eye_i
    A = a_scr[...]
    x = jnp.concatenate([t, lab, img], axis=0)
    plist = [
        (w0_ref, as0_ref, ad0_ref, b0_ref, g0_ref, be0_ref),
        (w1_ref, as1_ref, ad1_ref, b1_ref, g1_ref, be1_ref),
        (w2_ref, as2_ref, ad2_ref, b2_ref, g2_ref, be2_ref),
    ]
    for (w, asw, adw, b, g, be) in plist:
        x = _gat_layer(x, A, w[...], asw[...], adw[...], b[...], g[...],
                       be[...], N, H)
    o_ref[0] = x[0:T, :]


def _run_fused(text_repr, label_repr, image_repr, params, interpret=False):
    B, T, H = text_repr.shape
    L = label_repr.shape[1]
    I = image_repr.shape[1]
    N = T + L + I
    out_ch = H // HEADS
    wspec = pl.BlockSpec((H, H), lambda b: (0, 0))
    aspec = pl.BlockSpec((HEADS, out_ch), lambda b: (0, 0))
    vspec = pl.BlockSpec((1, H), lambda b: (0, 0))
    in_specs = [
        pl.BlockSpec((1, T, H), lambda b: (b, 0, 0)),
        pl.BlockSpec((1, L, H), lambda b: (b, 0, 0)),
        pl.BlockSpec((1, I, H), lambda b: (b, 0, 0)),
    ]
    args = [text_repr, label_repr, image_repr]
    for (W, a_s, a_d, b, g, be) in params:
        in_specs += [wspec, aspec, aspec, vspec, vspec, vspec]
        args += [W, a_s, a_d, b.reshape(1, -1), g.reshape(1, -1),
                 be.reshape(1, -1)]
    return pl.pallas_call(
        functools.partial(_fused_kernel, T=T, L=L, I=I, H=H),
        grid=(B,),
        in_specs=in_specs,
        out_specs=pl.BlockSpec((1, T, H), lambda b: (b, 0, 0)),
        out_shape=jax.ShapeDtypeStruct((B, T, H), jnp.float32),
        scratch_shapes=[pltpu.VMEM((N, N), jnp.float32)],
        interpret=interpret,
    )(*args)


def _run(text_repr, label_repr, image_repr, params, interpret=False):
    B, T, H = text_repr.shape
    x = jnp.concatenate([text_repr, label_repr, image_repr], axis=1)
    A = _build_A(text_repr, label_repr, image_repr, interpret=interpret)
    for (W, a_s, a_d, b, g, be) in params:
        x = _layer(x, A, W, a_s, a_d, b.reshape(1, -1), g.reshape(1, -1),
                   be.reshape(1, -1), interpret=interpret)
    return x[:, :T, :]


def kernel(text_repr, label_repr, image_repr,
           W0, as0, ad0, b0, g0, be0,
           W1, as1, ad1, b1, g1, be1,
           W2, as2, ad2, b2, g2, be2):
    params = [
        (W0, as0, ad0, b0, g0, be0),
        (W1, as1, ad1, b1, g1, be1),
        (W2, as2, ad2, b2, g2, be2),
    ]
    return _run_fused(text_repr, label_repr, image_repr, params)
